# initial kernel scaffold (unmeasured)
import jax
import jax.numpy as jnp
from jax import lax
from jax.experimental import pallas as pl
from jax.experimental.pallas import tpu as pltpu


def kernel(x, W, labels):
    T, D = x.shape
    _, Vs = W.shape
    labels2 = labels.reshape(T, 1)

    def body(x_ref, w_ref, lab_ref, out_ref, stats_tx, stats_rx, send_sem, recv_sem):
        my_x = lax.axis_index("x")
        my_y = lax.axis_index("y")

        xb = x_ref[:].astype(jnp.bfloat16)
        wb = w_ref[:].astype(jnp.bfloat16)
        logits = jnp.dot(xb, wb, preferred_element_type=jnp.float32)

        m = jnp.max(logits, axis=1, keepdims=True)
        s = jnp.sum(jnp.exp(logits - m), axis=1, keepdims=True)
        col = lax.broadcasted_iota(jnp.int32, (T, Vs), 1)
        lab_local = lab_ref[:] - my_x * Vs
        ll = jnp.sum(
            jnp.where(col == lab_local, logits, 0.0), axis=1, keepdims=True
        )

        stats_tx[:, 0:1] = m
        stats_tx[:, 1:2] = s
        stats_tx[:, 2:3] = ll

        rdma = pltpu.make_async_remote_copy(
            src_ref=stats_tx,
            dst_ref=stats_rx,
            send_sem=send_sem,
            recv_sem=recv_sem,
            device_id=(1 - my_x, my_y),
            device_id_type=pl.DeviceIdType.MESH,
        )
        rdma.start()
        rdma.wait()

        m_r = stats_rx[:, 0:1]
        s_r = stats_rx[:, 1:2]
        ll_r = stats_rx[:, 2:3]
        m_g = jnp.maximum(m, m_r)
        s_g = s * jnp.exp(m - m_g) + s_r * jnp.exp(m_r - m_g)
        nll = (m_g + jnp.log(s_g)) - (ll + ll_r)
        out_ref[:] = nll[:, 0]

    return pl.pallas_call(
        body,
        out_shape=jax.ShapeDtypeStruct((T,), jnp.float32),
        in_specs=[
            pl.BlockSpec(memory_space=pltpu.VMEM),
            pl.BlockSpec(memory_space=pltpu.VMEM),
            pl.BlockSpec(memory_space=pltpu.VMEM),
        ],
        out_specs=pl.BlockSpec(memory_space=pltpu.VMEM),
        scratch_shapes=[
            pltpu.VMEM((T, 8), jnp.float32),
            pltpu.VMEM((T, 8), jnp.float32),
            pltpu.SemaphoreType.DMA,
            pltpu.SemaphoreType.DMA,
        ],
        compiler_params=pltpu.CompilerParams(collective_id=0),
    )(x, W, labels2)


# baseline (device time: 31756 ns/iter reference)
import jax
import jax.numpy as jnp
from jax import lax
from jax.experimental import pallas as pl
from jax.experimental.pallas import tpu as pltpu

CHUNK = 2048


def kernel(x, W, labels):
    T, D = x.shape
    _, Vs = W.shape
    n_chunks = Vs // CHUNK
    labels2 = labels.reshape(T, 1)

    def body(
        x_ref,
        w_ref,
        lab_ref,
        out_ref,
        xb_ref,
        m_ref,
        s_ref,
        ll_ref,
        stats_tx,
        stats_rx,
        send_sem,
        recv_sem,
    ):
        i = pl.program_id(0)
        my_x = lax.axis_index("x")
        my_y = lax.axis_index("y")

        @pl.when(i == 0)
        def _init():
            xb_ref[...] = x_ref[...].astype(jnp.bfloat16)
            m_ref[...] = jnp.full((T, 1), -jnp.inf, jnp.float32)
            s_ref[...] = jnp.zeros((T, 1), jnp.float32)
            ll_ref[...] = jnp.zeros((T, 1), jnp.float32)

        wb = w_ref[...].astype(jnp.bfloat16)
        logits = jnp.dot(xb_ref[...], wb, preferred_element_type=jnp.float32)

        cm = jnp.max(logits, axis=1, keepdims=True)
        m_old = m_ref[...]
        m_new = jnp.maximum(m_old, cm)
        s_ref[...] = s_ref[...] * jnp.exp(m_old - m_new) + jnp.sum(
            jnp.exp(logits - m_new), axis=1, keepdims=True
        )
        m_ref[...] = m_new

        col = lax.broadcasted_iota(jnp.int32, (T, CHUNK), 1)
        lab_local = lab_ref[...] - (my_x * Vs + i * CHUNK)
        ll_ref[...] += jnp.sum(
            jnp.where(col == lab_local, logits, 0.0), axis=1, keepdims=True
        )

        @pl.when(i == n_chunks - 1)
        def _exchange():
            m = m_ref[...]
            s = s_ref[...]
            ll = ll_ref[...]
            stats_tx[:, 0:1] = m
            stats_tx[:, 1:2] = s
            stats_tx[:, 2:3] = ll

            rdma = pltpu.make_async_remote_copy(
                src_ref=stats_tx,
                dst_ref=stats_rx,
                send_sem=send_sem,
                recv_sem=recv_sem,
                device_id=(1 - my_x, my_y),
                device_id_type=pl.DeviceIdType.MESH,
            )
            rdma.start()
            rdma.wait()

            m_r = stats_rx[:, 0:1]
            s_r = stats_rx[:, 1:2]
            ll_r = stats_rx[:, 2:3]
            m_g = jnp.maximum(m, m_r)
            s_g = s * jnp.exp(m - m_g) + s_r * jnp.exp(m_r - m_g)
            nll = (m_g + jnp.log(s_g)) - (ll + ll_r)
            out_ref[...] = nll[:, 0]

    return pl.pallas_call(
        body,
        grid=(n_chunks,),
        out_shape=jax.ShapeDtypeStruct((T,), jnp.float32),
        in_specs=[
            pl.BlockSpec((T, D), lambda i: (0, 0), memory_space=pltpu.VMEM),
            pl.BlockSpec((D, CHUNK), lambda i: (0, i), memory_space=pltpu.VMEM),
            pl.BlockSpec((T, 1), lambda i: (0, 0), memory_space=pltpu.VMEM),
        ],
        out_specs=pl.BlockSpec((T,), lambda i: (0,), memory_space=pltpu.VMEM),
        scratch_shapes=[
            pltpu.VMEM((T, D), jnp.bfloat16),
            pltpu.VMEM((T, 1), jnp.float32),
            pltpu.VMEM((T, 1), jnp.float32),
            pltpu.VMEM((T, 1), jnp.float32),
            pltpu.VMEM((T, 8), jnp.float32),
            pltpu.VMEM((T, 8), jnp.float32),
            pltpu.SemaphoreType.DMA,
            pltpu.SemaphoreType.DMA,
        ],
        compiler_params=pltpu.CompilerParams(
            dimension_semantics=("arbitrary",),
        ),
    )(x, W, labels2)


# device time: 20331 ns/iter; 1.5619x vs baseline; 1.5619x over previous
import jax
import jax.numpy as jnp
from jax import lax
from jax.experimental import pallas as pl
from jax.experimental.pallas import tpu as pltpu

CHUNK = 1024
N_CHUNKS = 4
N_PHASES = 2
PER_PHASE = N_CHUNKS // N_PHASES


def kernel(x, W, labels):
    T, D = x.shape
    _, Vs = W.shape
    Vh = Vs // 2
    assert Vh == N_CHUNKS * CHUNK

    def body(
        x_ref,
        lab_ref,
        w_hbm,
        out_hbm,
        bufs,
        tx,
        rx,
        nll_vmem,
        copy_sems,
        out_sem,
        send_sems,
        recv_sems,
    ):
        my_x = lax.axis_index("x")
        my_y = lax.axis_index("y")
        base = my_y * Vh

        peers = [
            (1 - my_x, 1 - my_y),
            (1 - my_x, my_y),
            (my_x, 1 - my_y),
        ]

        barrier_sem = pltpu.get_barrier_semaphore()
        for p in peers:
            pl.semaphore_signal(
                barrier_sem, inc=1,
                device_id=p, device_id_type=pl.DeviceIdType.MESH,
            )

        def chunk_copy(i):
            return pltpu.make_async_copy(
                w_hbm.at[:, pl.ds(base + i * CHUNK, CHUNK)],
                bufs.at[i],
                copy_sems.at[i],
            )

        for i in range(N_CHUNKS):
            chunk_copy(i).start()

        ones_c = jnp.ones((CHUNK, 1), jnp.float32)
        i0 = lax.broadcasted_iota(jnp.int32, (T, T), 0)
        i1 = lax.broadcasted_iota(jnp.int32, (T, T), 1)
        eye_b = (i0 == i1).astype(jnp.bfloat16)
        eye_f = (i0 == i1).astype(jnp.float32)

        lab_row = jnp.reshape(lab_ref[...], (1, T)).astype(jnp.float32)
        lab_col = lax.dot_general(
            eye_f, lab_row, (((1,), (1,)), ((), ())),
            preferred_element_type=jnp.float32,
        )
        lab_local = lab_col - (my_x * Vs + base)

        rdmas = []
        for ph in range(N_PHASES):
            s_row = jnp.zeros((1, T), jnp.float32)
            ll = jnp.zeros((T, 1), jnp.float32)
            for j in range(PER_PHASE):
                i = ph * PER_PHASE + j
                chunk_copy(i).wait()
                logits = jnp.dot(
                    x_ref[...], bufs[i], preferred_element_type=jnp.float32
                )
                e = jnp.exp(logits)
                s_row = s_row + lax.dot_general(
                    ones_c, e, (((0,), (1,)), ((), ())),
                    preferred_element_type=jnp.float32,
                )
                rel = lab_local - i * CHUNK
                for g in range(CHUNK // 128):
                    blk = logits[:, g * 128 : (g + 1) * 128]
                    idxg = jnp.clip(rel - g * 128, 0.0, 127.0).astype(jnp.int32)
                    got = jnp.take_along_axis(blk, idxg, axis=1)
                    ll = ll + jnp.where(
                        (rel >= g * 128) & (rel < (g + 1) * 128), got, 0.0
                    )

            ll_row = lax.dot_general(
                ll.astype(jnp.bfloat16), eye_b, (((0,), (0,)), ((), ())),
                preferred_element_type=jnp.float32,
            )

            tx[ph, 0:1, :] = s_row
            tx[ph, 1:2, :] = ll_row

            if ph == 0:
                pl.semaphore_wait(barrier_sem, 3)

            for k, p in enumerate(peers):
                r = pltpu.make_async_remote_copy(
                    src_ref=tx.at[ph],
                    dst_ref=rx.at[k, ph],
                    send_sem=send_sems.at[k, ph],
                    recv_sem=recv_sems.at[k, ph],
                    device_id=p,
                    device_id_type=pl.DeviceIdType.MESH,
                )
                r.start()
                rdmas.append(r)

        for r in rdmas:
            r.wait()

        tot = tx[0] + tx[1]
        for k in range(3):
            for ph in range(N_PHASES):
                tot = tot + rx[k, ph]
        nll_vmem[...] = jnp.log(tot[0:1, :]) - tot[1:2, :]

        out_copy = pltpu.make_async_copy(
            nll_vmem, out_hbm, out_sem
        )
        out_copy.start()
        out_copy.wait()

    return pl.pallas_call(
        body,
        out_shape=jax.ShapeDtypeStruct((1, T), jnp.float32),
        in_specs=[
            pl.BlockSpec(memory_space=pltpu.VMEM),
            pl.BlockSpec(memory_space=pltpu.VMEM),
            pl.BlockSpec(memory_space=pl.ANY),
        ],
        out_specs=pl.BlockSpec(memory_space=pl.ANY),
        scratch_shapes=[
            pltpu.VMEM((N_CHUNKS, D, CHUNK), jnp.float32),
            pltpu.VMEM((N_PHASES, 8, T), jnp.float32),
            pltpu.VMEM((3, N_PHASES, 8, T), jnp.float32),
            pltpu.VMEM((1, T), jnp.float32),
            pltpu.SemaphoreType.DMA((N_CHUNKS,)),
            pltpu.SemaphoreType.DMA,
            pltpu.SemaphoreType.DMA((3, N_PHASES)),
            pltpu.SemaphoreType.DMA((3, N_PHASES)),
        ],
        compiler_params=pltpu.CompilerParams(
            vmem_limit_bytes=64 * 1024 * 1024,
            collective_id=0,
        ),
    )(x, labels, W).reshape(T)
